# Initial kernel scaffold; baseline (speedup 1.0000x reference)
import jax
import jax.numpy as jnp
from jax.experimental import pallas as pl


def kernel(x, edge_index, edge_attr, W0, b0, W1, b1, W2, b2):
    raise NotImplementedError("write your pallas kernel here")


# speedup vs baseline: 6.7392x; 6.7392x over previous
import functools

import jax
import jax.numpy as jnp
from jax import lax
from jax.experimental import pallas as pl
from jax.experimental.pallas import tpu as pltpu
from jax.experimental.pallas import tpu_sc as plsc

N_NODES = 10000
N_EDGES = 320000
D = 128

NC = 2
NS = 16
NW = NC * NS

N_PAD = 10240
CH = 128
EPW = 10240
NCHUNK = EPW // CH
E_PAD = NW * EPW
ROWS_T = N_PAD // NS

_MESH = plsc.VectorSubcoreMesh(
    core_axis_name="c", subcore_axis_name="s", num_cores=NC, num_subcores=NS
)


@functools.partial(
    pl.kernel,
    out_type=jax.ShapeDtypeStruct((NC, N_PAD, 16), jnp.float32),
    mesh=_MESH,
    scratch_types=[
        pltpu.VMEM((NCHUNK, CH), jnp.int32),
        pltpu.VMEM((EPW,), jnp.float32),
        pltpu.VMEM((CH, 16), jnp.float32),
        pltpu.VMEM_SHARED((N_PAD, 16), jnp.float32),
    ],
)
def _sc_deg(dst_hbm, ew_hbm, zdeg_hbm, out_hbm, dst_v, ew_v, rows_v, deg_sh):
    cid = lax.axis_index("c")
    sid = lax.axis_index("s")
    wid = sid * NC + cid

    r0 = sid * ROWS_T
    pltpu.sync_copy(zdeg_hbm.at[pl.ds(r0, ROWS_T)], deg_sh.at[pl.ds(r0, ROWS_T)])

    pltpu.sync_copy(dst_hbm.at[pl.ds(wid * NCHUNK, NCHUNK)], dst_v)
    pltpu.sync_copy(ew_hbm.at[pl.ds(wid * EPW, EPW)], ew_v)
    plsc.subcore_barrier()

    def chunk_body(g, _):
        def grp_body(k, _):
            ewv = ew_v[pl.ds(g * CH + k * 16, 16)]
            for e16 in range(16):
                w = ewv.at[jnp.full((16,), e16, jnp.int32)].get(
                    mode="promise_in_bounds")
                rows_v.at[k * 16 + e16][pl.ds(0, 16)] = w
            return 0
        lax.fori_loop(0, CH // 16, grp_body, 0)
        pltpu.sync_copy(rows_v, deg_sh.at[dst_v.at[g]], add=True)
        return 0

    lax.fori_loop(0, NCHUNK, chunk_body, 0)
    plsc.subcore_barrier()

    pltpu.sync_copy(deg_sh.at[pl.ds(r0, ROWS_T)], out_hbm.at[cid, pl.ds(r0, ROWS_T)])


@functools.partial(
    pl.kernel,
    out_type=jax.ShapeDtypeStruct((NC, N_PAD, D), jnp.float32),
    mesh=_MESH,
    scratch_types=[
        pltpu.VMEM((NCHUNK, CH), jnp.int32),
        pltpu.VMEM((NCHUNK, CH), jnp.int32),
        pltpu.VMEM((EPW,), jnp.float32),
        pltpu.VMEM((CH, D), jnp.float32),
        pltpu.VMEM_SHARED((N_PAD, D), jnp.float32),
        pltpu.SemaphoreType.DMA,
    ],
)
def _sc_agg(g_hbm, src_hbm, dst_hbm, ew_hbm, zacc_hbm, out_hbm,
            src_v, dst_v, ew_v, rows_v, acc_sh, sem):
    cid = lax.axis_index("c")
    sid = lax.axis_index("s")
    wid = sid * NC + cid

    r0 = sid * ROWS_T
    pltpu.sync_copy(zacc_hbm.at[pl.ds(r0, ROWS_T)], acc_sh.at[pl.ds(r0, ROWS_T)])

    pltpu.sync_copy(src_hbm.at[pl.ds(wid * NCHUNK, NCHUNK)], src_v)
    pltpu.sync_copy(dst_hbm.at[pl.ds(wid * NCHUNK, NCHUNK)], dst_v)
    pltpu.sync_copy(ew_hbm.at[pl.ds(wid * EPW, EPW)], ew_v)
    plsc.subcore_barrier()

    def chunk_body(g, _):
        pltpu.async_copy(g_hbm.at[src_v.at[g]], rows_v, sem).wait()

        def grp_body(k, _):
            ewv = ew_v[pl.ds(g * CH + k * 16, 16)]
            for e16 in range(16):
                w = ewv.at[jnp.full((16,), e16, jnp.int32)].get(
                    mode="promise_in_bounds")
                row = rows_v.at[k * 16 + e16]
                for j in range(D // 16):
                    sl = pl.ds(j * 16, 16)
                    row[sl] = row[sl] * w
            return 0
        lax.fori_loop(0, CH // 16, grp_body, 0)

        pltpu.sync_copy(rows_v, acc_sh.at[dst_v.at[g]], add=True)
        return 0

    lax.fori_loop(0, NCHUNK, chunk_body, 0)
    plsc.subcore_barrier()

    pltpu.sync_copy(acc_sh.at[pl.ds(r0, ROWS_T)], out_hbm.at[cid, pl.ds(r0, ROWS_T)])


_BLK = 1280
_GRID = N_PAD // _BLK


def _dinv_block(degp):
    return lax.rsqrt(degp[0, :] + degp[1, :] + 1.0)[:, None]


def _tc_prep_body(degp_ref, x_ref, w_ref, g_ref):
    h = jnp.dot(x_ref[...], w_ref[...], preferred_element_type=jnp.float32)
    g_ref[...] = _dinv_block(degp_ref[...]) * h


def _tc_mid_body(degp_ref, accp_ref, g_ref, b_ref, w_ref, out_ref):
    dinv = _dinv_block(degp_ref[...])
    pre = dinv * (accp_ref[0] + accp_ref[1] + g_ref[...]) + b_ref[0][None, :]
    h = jnp.maximum(pre, 0.0)
    out_ref[...] = dinv * jnp.dot(h, w_ref[...], preferred_element_type=jnp.float32)


def _tc_final_body(degp_ref, accp_ref, g_ref, b_ref, out_ref):
    dinv = _dinv_block(degp_ref[...])
    pre = dinv * (accp_ref[0] + accp_ref[1] + g_ref[...]) + b_ref[0][None, :]
    out_ref[...] = jnp.maximum(pre, 0.0)


_degp_spec = pl.BlockSpec((NC, _BLK), lambda i: (0, i))
_rows_spec = pl.BlockSpec((_BLK, D), lambda i: (i, 0))
_accp_spec = pl.BlockSpec((NC, _BLK, D), lambda i: (0, i, 0))
_w_spec = pl.BlockSpec((D, D), lambda i: (0, 0))
_b_spec = pl.BlockSpec((1, D), lambda i: (0, 0))
_f32 = jnp.float32

_tc_prep = pl.pallas_call(
    _tc_prep_body,
    grid=(_GRID,),
    in_specs=[_degp_spec, _rows_spec, _w_spec],
    out_specs=_rows_spec,
    out_shape=jax.ShapeDtypeStruct((N_PAD, D), _f32),
)

_tc_mid = pl.pallas_call(
    _tc_mid_body,
    grid=(_GRID,),
    in_specs=[_degp_spec, _accp_spec, _rows_spec, _b_spec, _w_spec],
    out_specs=_rows_spec,
    out_shape=jax.ShapeDtypeStruct((N_PAD, D), _f32),
)

_tc_final = pl.pallas_call(
    _tc_final_body,
    grid=(_GRID,),
    in_specs=[_degp_spec, _accp_spec, _rows_spec, _b_spec],
    out_specs=_rows_spec,
    out_shape=jax.ShapeDtypeStruct((N_PAD, D), _f32),
)


def kernel(x, edge_index, edge_attr, W0, b0, W1, b1, W2, b2):
    src = edge_index[0].astype(jnp.int32)
    dst = edge_index[1].astype(jnp.int32)
    ew = edge_attr.astype(jnp.float32)

    pad = E_PAD - N_EDGES
    src_p = jnp.pad(src, (0, pad)).reshape(E_PAD // CH, CH)
    dst_p = jnp.pad(dst, (0, pad)).reshape(E_PAD // CH, CH)
    ew_p = jnp.pad(ew, (0, pad))

    x_p = jnp.pad(x, ((0, N_PAD - N_NODES), (0, 0)))
    zdeg = jnp.zeros((N_PAD, 16), _f32)
    zacc = jnp.zeros((N_PAD, D), _f32)

    degp = _sc_deg(dst_p, ew_p, zdeg)[:, :, 0]
    g0 = _tc_prep(degp, x_p, W0)
    acc0 = _sc_agg(g0, src_p, dst_p, ew_p, zacc)
    g1 = _tc_mid(degp, acc0, g0, b0.reshape(1, D), W1)
    acc1 = _sc_agg(g1, src_p, dst_p, ew_p, zacc)
    g2 = _tc_mid(degp, acc1, g1, b1.reshape(1, D), W2)
    acc2 = _sc_agg(g2, src_p, dst_p, ew_p, zacc)
    out = _tc_final(degp, acc2, g2, b2.reshape(1, D))
    return out[:N_NODES]
